# Initial kernel scaffold; baseline (speedup 1.0000x reference)
import jax
import jax.numpy as jnp
from jax.experimental import pallas as pl


def kernel(x, edge_index, graph_ids, W_emb, W1, b1, W2, b2, W3, b3, Wp1, Wp2, bp2):
    raise NotImplementedError("write your pallas kernel here")


# speedup vs baseline: 4.4141x; 4.4141x over previous
import functools

import jax
import jax.numpy as jnp
from jax import lax
from jax.experimental import pallas as pl
from jax.experimental.pallas import tpu as pltpu
from jax.experimental.pallas import tpu_sc as plsc

N = 50000
E = 800000
D_N = 35
IN_CH = 8
HID = 64
EMB = 32
G = 64

NC = 2
NS = 16
CH = 128
SC_PER_SUP = 8
NCHUNK = 6400
NSUP = NCHUNK // SC_PER_SUP
E_PAD = NCHUNK * CH
NPAD = 50048
ACC_ROWS = NPAD
ZROWS = ACC_ROWS // NS
OROWS = NPAD // NS
BN = 400
GRID = N // BN

@functools.cache
def _sc_mesh():
    return plsc.VectorSubcoreMesh(
        core_axis_name="c", subcore_axis_name="s",
        num_cores=NC, num_subcores=NS)


def _seg_body_d8(table, srcp, dstp, zeros, out, sbuf, dbuf, rows, acc, gsem):
    c = lax.axis_index("c")
    s = lax.axis_index("s")
    pltpu.sync_copy(zeros.at[pl.ds(0, ZROWS)], acc.at[pl.ds(s * ZROWS, ZROWS)])
    plsc.subcore_barrier()

    sups_per_tile = NSUP // NC // NS
    base = c * (NSUP // NC) + s * sups_per_tile

    def step(j, carry):
        q = base + j
        pltpu.sync_copy(srcp.at[q], sbuf)
        pltpu.sync_copy(dstp.at[q], dbuf)
        for r in range(SC_PER_SUP):
            pltpu.async_copy(table.at[sbuf.at[r]], rows, gsem).wait()
            pltpu.sync_copy(rows, acc.at[dbuf.at[r]], add=True)
        return carry

    lax.fori_loop(0, sups_per_tile, step, 0)
    plsc.subcore_barrier()
    pltpu.sync_copy(acc.at[pl.ds(s * OROWS, OROWS)],
                    out.at[c, pl.ds(s * OROWS, OROWS)])


def _seg_body_d32(table, srcp, dstp, zeros, out, sbuf, ibuf, dbuf, rows, acc,
                  gsem):
    c = lax.axis_index("c")
    s = lax.axis_index("s")
    pltpu.sync_copy(zeros.at[pl.ds(0, ZROWS)], acc.at[pl.ds(s * ZROWS, ZROWS)])
    plsc.subcore_barrier()

    sups_per_tile = NSUP // NS
    base = s * sups_per_tile
    off = c * N

    def step(j, carry):
        q = base + j
        pltpu.sync_copy(srcp.at[q], sbuf)
        pltpu.sync_copy(dstp.at[q], dbuf)
        for r in range(SC_PER_SUP):
            for t in range(CH // 16):
                ibuf[r, pl.ds(t * 16, 16)] = sbuf[r, pl.ds(t * 16, 16)] + off
            pltpu.async_copy(table.at[ibuf.at[r]], rows, gsem).wait()
            pltpu.sync_copy(rows, acc.at[dbuf.at[r]], add=True)
        return carry

    lax.fori_loop(0, sups_per_tile, step, 0)
    plsc.subcore_barrier()
    pltpu.sync_copy(acc.at[pl.ds(s * OROWS, OROWS)],
                    out.at[c, pl.ds(s * OROWS, OROWS)])


@functools.cache
def _segsum_d8():
    return pl.kernel(
        _seg_body_d8,
        out_type=jax.ShapeDtypeStruct((NC, NPAD, IN_CH), jnp.float32),
        mesh=_sc_mesh(),
        compiler_params=pltpu.CompilerParams(use_tc_tiling_on_sc=False),
        scratch_types=[
            pltpu.VMEM((SC_PER_SUP, CH), jnp.int32),
            pltpu.VMEM((SC_PER_SUP, CH), jnp.int32),
            pltpu.VMEM((CH, IN_CH), jnp.float32),
            pltpu.VMEM_SHARED((ACC_ROWS, IN_CH), jnp.float32),
            pltpu.SemaphoreType.DMA,
        ],
    )


@functools.cache
def _segsum_d32():
    return pl.kernel(
        _seg_body_d32,
        out_type=jax.ShapeDtypeStruct((NC, NPAD, EMB), jnp.float32),
        mesh=_sc_mesh(),
        compiler_params=pltpu.CompilerParams(use_tc_tiling_on_sc=False),
        scratch_types=[
            pltpu.VMEM((SC_PER_SUP, CH), jnp.int32),
            pltpu.VMEM((SC_PER_SUP, CH), jnp.int32),
            pltpu.VMEM((SC_PER_SUP, CH), jnp.int32),
            pltpu.VMEM((CH, EMB), jnp.float32),
            pltpu.VMEM_SHARED((ACC_ROWS, EMB), jnp.float32),
            pltpu.SemaphoreType.DMA,
        ],
    )


def _emb_body(x_ref, w_ref, o_ref):
    o_ref[...] = jnp.dot(x_ref[...], w_ref[...],
                         preferred_element_type=jnp.float32)


def _mlp1_body(m_ref, w_ref, b_ref, o_ref):
    m = m_ref[0] + m_ref[1]
    h = jax.nn.relu(jnp.dot(m, w_ref[...],
                            preferred_element_type=jnp.float32) + b_ref[...])
    o_ref[0] = h[:, :EMB]
    o_ref[1] = h[:, EMB:]


def _mlp_res_body(m_ref, hprev_ref, w_ref, b_ref, o_ref):
    w = w_ref[...]
    acc = jnp.dot(m_ref[0], w[:EMB], preferred_element_type=jnp.float32)
    acc += jnp.dot(m_ref[1], w[EMB:], preferred_element_type=jnp.float32)
    h = jax.nn.relu(acc + b_ref[...])
    o_ref[0] = h[:, :EMB] + hprev_ref[0]
    o_ref[1] = h[:, EMB:] + hprev_ref[1]


def _final_body(m_ref, hprev_ref, gid_ref, w_ref, b_ref, wp1_ref, wp2_ref,
                bp2_ref, node_ref, graph_ref, pred_ref):
    i = pl.program_id(0)
    w = w_ref[...]
    acc = jnp.dot(m_ref[0], w[:EMB], preferred_element_type=jnp.float32)
    acc += jnp.dot(m_ref[1], w[EMB:], preferred_element_type=jnp.float32)
    h = jax.nn.relu(acc + b_ref[...])
    h = h + jnp.concatenate([hprev_ref[0], hprev_ref[1]], axis=1)
    node_ref[...] = h
    gid = gid_ref[0]
    seg = lax.broadcasted_iota(jnp.int32, (G, BN), 0)
    onehot = jnp.where(seg == gid, 1.0, 0.0).astype(jnp.float32)
    contrib = jnp.dot(onehot, h, preferred_element_type=jnp.float32)

    @pl.when(i == 0)
    def _init():
        graph_ref[...] = contrib

    @pl.when(i != 0)
    def _acc():
        graph_ref[...] += contrib

    @pl.when(i == GRID - 1)
    def _head():
        eg = graph_ref[...]
        p = jax.nn.relu(jnp.dot(eg, wp1_ref[...],
                                preferred_element_type=jnp.float32))
        pred_ref[...] = jnp.dot(p, wp2_ref[...],
                                preferred_element_type=jnp.float32) + bp2_ref[...]


def _emb_call(x, W_emb):
    return pl.pallas_call(
        _emb_body,
        grid=(GRID,),
        in_specs=[
            pl.BlockSpec((BN, D_N), lambda i: (i, 0)),
            pl.BlockSpec((D_N, IN_CH), lambda i: (0, 0)),
        ],
        out_specs=pl.BlockSpec((BN, IN_CH), lambda i: (i, 0)),
        out_shape=jax.ShapeDtypeStruct((N, IN_CH), jnp.float32),
    )(x, W_emb)


def _mlp1_call(m1p, W1, b1):
    return pl.pallas_call(
        _mlp1_body,
        grid=(GRID,),
        in_specs=[
            pl.BlockSpec((NC, BN, IN_CH), lambda i: (0, i, 0)),
            pl.BlockSpec((IN_CH, HID), lambda i: (0, 0)),
            pl.BlockSpec((HID,), lambda i: (0,)),
        ],
        out_specs=pl.BlockSpec((NC, BN, EMB), lambda i: (0, i, 0)),
        out_shape=jax.ShapeDtypeStruct((NC, N, EMB), jnp.float32),
    )(m1p, W1, b1)


def _mlp_res_call(m, hprev, W, b):
    return pl.pallas_call(
        _mlp_res_body,
        grid=(GRID,),
        in_specs=[
            pl.BlockSpec((NC, BN, EMB), lambda i: (0, i, 0)),
            pl.BlockSpec((NC, BN, EMB), lambda i: (0, i, 0)),
            pl.BlockSpec((HID, HID), lambda i: (0, 0)),
            pl.BlockSpec((HID,), lambda i: (0,)),
        ],
        out_specs=pl.BlockSpec((NC, BN, EMB), lambda i: (0, i, 0)),
        out_shape=jax.ShapeDtypeStruct((NC, N, EMB), jnp.float32),
    )(m, hprev, W, b)


def _final_call(m3, h2, gid3, W3, b3, Wp1, Wp2, bp2):
    return pl.pallas_call(
        _final_body,
        grid=(GRID,),
        in_specs=[
            pl.BlockSpec((NC, BN, EMB), lambda i: (0, i, 0)),
            pl.BlockSpec((NC, BN, EMB), lambda i: (0, i, 0)),
            pl.BlockSpec((1, 1, BN), lambda i: (i, 0, 0)),
            pl.BlockSpec((HID, HID), lambda i: (0, 0)),
            pl.BlockSpec((HID,), lambda i: (0,)),
            pl.BlockSpec((HID, EMB), lambda i: (0, 0)),
            pl.BlockSpec((EMB, 1), lambda i: (0, 0)),
            pl.BlockSpec((1,), lambda i: (0,)),
        ],
        out_specs=[
            pl.BlockSpec((BN, HID), lambda i: (i, 0)),
            pl.BlockSpec((G, HID), lambda i: (0, 0)),
            pl.BlockSpec((G, 1), lambda i: (0, 0)),
        ],
        out_shape=[
            jax.ShapeDtypeStruct((N, HID), jnp.float32),
            jax.ShapeDtypeStruct((G, HID), jnp.float32),
            jax.ShapeDtypeStruct((G, 1), jnp.float32),
        ],
    )(m3, h2, gid3, W3, b3, Wp1, Wp2, bp2)


@functools.partial(jax.jit)
def kernel(x, edge_index, graph_ids, W_emb, W1, b1, W2, b2, W3, b3, Wp1, Wp2,
           bp2):
    src = edge_index[0]
    dst = edge_index[1]
    pad = E_PAD - E
    srcp = jnp.concatenate([src, jnp.zeros((pad,), jnp.int32)]) \
              .reshape(NSUP, SC_PER_SUP, CH)
    dstp = jnp.concatenate([dst, jnp.full((pad,), N, jnp.int32)]) \
              .reshape(NSUP, SC_PER_SUP, CH)
    zeros8 = jnp.zeros((ZROWS, IN_CH), jnp.float32)
    zeros32 = jnp.zeros((ZROWS, EMB), jnp.float32)
    gid3 = graph_ids.reshape(GRID, 1, BN)

    h0 = _emb_call(x, W_emb)
    m1p = _segsum_d8()(h0, srcp, dstp, zeros8)
    h1 = _mlp1_call(m1p, W1, b1)
    m2 = _segsum_d32()(h1.reshape(NC * N, EMB), srcp, dstp, zeros32)
    h2 = _mlp_res_call(m2, h1, W2, b2)
    m3 = _segsum_d32()(h2.reshape(NC * N, EMB), srcp, dstp, zeros32)
    emb_node, emb_graph, pred = _final_call(m3, h2, gid3, W3, b3, Wp1, Wp2,
                                            bp2)
    return (emb_node, emb_graph, pred)


# speedup vs baseline: 5.4221x; 1.2284x over previous
import functools

import jax
import jax.numpy as jnp
from jax import lax
from jax.experimental import pallas as pl
from jax.experimental.pallas import tpu as pltpu
from jax.experimental.pallas import tpu_sc as plsc

N = 50000
E = 800000
D_N = 35
IN_CH = 8
HID = 64
EMB = 32
G = 64

NC = 2
NS = 16
CH = 128
SC_PER_SUP = 8
NCHUNK = 6400
NSUP = NCHUNK // SC_PER_SUP
E_PAD = NCHUNK * CH
NPAD = 50048
ACC_ROWS = NPAD
ZROWS = ACC_ROWS // NS
OROWS = NPAD // NS
BN = 400
GRID = N // BN

@functools.cache
def _sc_mesh():
    return plsc.VectorSubcoreMesh(
        core_axis_name="c", subcore_axis_name="s",
        num_cores=NC, num_subcores=NS)


def _seg_body_d8(table, srcp, dstp, zeros, out, sbuf, dbuf, rows, acc,
                 gsem0, gsem1):
    gsems = (gsem0, gsem1)
    c = lax.axis_index("c")
    s = lax.axis_index("s")
    pltpu.sync_copy(zeros.at[pl.ds(0, ZROWS)], acc.at[pl.ds(s * ZROWS, ZROWS)])
    plsc.subcore_barrier()

    sups_per_tile = NSUP // NC // NS
    base = c * (NSUP // NC) + s * sups_per_tile

    def step(j, carry):
        q = base + j
        pltpu.sync_copy(srcp.at[q], sbuf)
        pltpu.sync_copy(dstp.at[q], dbuf)
        cps = [pltpu.async_copy(table.at[sbuf.at[r]], rows.at[r % 2],
                                gsems[r % 2])
               for r in range(2)]
        for r in range(SC_PER_SUP):
            cps[r].wait()
            pltpu.sync_copy(rows.at[r % 2], acc.at[dbuf.at[r]], add=True)
            if r + 2 < SC_PER_SUP:
                cps.append(pltpu.async_copy(
                    table.at[sbuf.at[r + 2]], rows.at[r % 2], gsems[r % 2]))
        return carry

    lax.fori_loop(0, sups_per_tile, step, 0)
    plsc.subcore_barrier()
    pltpu.sync_copy(acc.at[pl.ds(s * OROWS, OROWS)],
                    out.at[c, pl.ds(s * OROWS, OROWS)])


def _seg_body_d32(table, srcp, dstp, zeros, out, sbuf, ibuf, dbuf, rows, acc,
                  gsem0, gsem1):
    gsems = (gsem0, gsem1)
    c = lax.axis_index("c")
    s = lax.axis_index("s")
    pltpu.sync_copy(zeros.at[pl.ds(0, ZROWS)], acc.at[pl.ds(s * ZROWS, ZROWS)])
    plsc.subcore_barrier()

    sups_per_tile = NSUP // NS
    base = s * sups_per_tile
    off = c * N

    def step(j, carry):
        q = base + j
        pltpu.sync_copy(srcp.at[q], sbuf)
        pltpu.sync_copy(dstp.at[q], dbuf)
        for r in range(SC_PER_SUP):
            for t in range(CH // 16):
                ibuf[r, pl.ds(t * 16, 16)] = sbuf[r, pl.ds(t * 16, 16)] + off
        cps = [pltpu.async_copy(table.at[ibuf.at[r]], rows.at[r % 2],
                                gsems[r % 2])
               for r in range(2)]
        for r in range(SC_PER_SUP):
            cps[r].wait()
            pltpu.sync_copy(rows.at[r % 2], acc.at[dbuf.at[r]], add=True)
            if r + 2 < SC_PER_SUP:
                cps.append(pltpu.async_copy(
                    table.at[ibuf.at[r + 2]], rows.at[r % 2], gsems[r % 2]))
        return carry

    lax.fori_loop(0, sups_per_tile, step, 0)
    plsc.subcore_barrier()
    pltpu.sync_copy(acc.at[pl.ds(s * OROWS, OROWS)],
                    out.at[c, pl.ds(s * OROWS, OROWS)])


@functools.cache
def _segsum_d8():
    return pl.kernel(
        _seg_body_d8,
        out_type=jax.ShapeDtypeStruct((NC, NPAD, IN_CH), jnp.float32),
        mesh=_sc_mesh(),
        compiler_params=pltpu.CompilerParams(use_tc_tiling_on_sc=False),
        scratch_types=[
            pltpu.VMEM((SC_PER_SUP, CH), jnp.int32),
            pltpu.VMEM((SC_PER_SUP, CH), jnp.int32),
            pltpu.VMEM((2, CH, IN_CH), jnp.float32),
            pltpu.VMEM_SHARED((ACC_ROWS, IN_CH), jnp.float32),
            pltpu.SemaphoreType.DMA,
            pltpu.SemaphoreType.DMA,
        ],
    )


@functools.cache
def _segsum_d32():
    return pl.kernel(
        _seg_body_d32,
        out_type=jax.ShapeDtypeStruct((NC, NPAD, EMB), jnp.float32),
        mesh=_sc_mesh(),
        compiler_params=pltpu.CompilerParams(use_tc_tiling_on_sc=False),
        scratch_types=[
            pltpu.VMEM((SC_PER_SUP, CH), jnp.int32),
            pltpu.VMEM((SC_PER_SUP, CH), jnp.int32),
            pltpu.VMEM((SC_PER_SUP, CH), jnp.int32),
            pltpu.VMEM((2, CH, EMB), jnp.float32),
            pltpu.VMEM_SHARED((ACC_ROWS, EMB), jnp.float32),
            pltpu.SemaphoreType.DMA,
            pltpu.SemaphoreType.DMA,
        ],
    )


def _emb_body(x_ref, w_ref, o_ref):
    o_ref[...] = jnp.dot(x_ref[...], w_ref[...],
                         preferred_element_type=jnp.float32)


def _mlp1_body(m_ref, w_ref, b_ref, o_ref):
    m = m_ref[0] + m_ref[1]
    h = jax.nn.relu(jnp.dot(m, w_ref[...],
                            preferred_element_type=jnp.float32) + b_ref[...])
    o_ref[0] = h[:, :EMB]
    o_ref[1] = h[:, EMB:]


def _mlp_res_body(m_ref, hprev_ref, w_ref, b_ref, o_ref):
    w = w_ref[...]
    acc = jnp.dot(m_ref[0], w[:EMB], preferred_element_type=jnp.float32)
    acc += jnp.dot(m_ref[1], w[EMB:], preferred_element_type=jnp.float32)
    h = jax.nn.relu(acc + b_ref[...])
    o_ref[0] = h[:, :EMB] + hprev_ref[0]
    o_ref[1] = h[:, EMB:] + hprev_ref[1]


def _final_body(m_ref, hprev_ref, gid_ref, w_ref, b_ref, wp1_ref, wp2_ref,
                bp2_ref, node_ref, graph_ref, pred_ref):
    i = pl.program_id(0)
    w = w_ref[...]
    acc = jnp.dot(m_ref[0], w[:EMB], preferred_element_type=jnp.float32)
    acc += jnp.dot(m_ref[1], w[EMB:], preferred_element_type=jnp.float32)
    h = jax.nn.relu(acc + b_ref[...])
    h = h + jnp.concatenate([hprev_ref[0], hprev_ref[1]], axis=1)
    node_ref[...] = h
    gid = gid_ref[0]
    seg = lax.broadcasted_iota(jnp.int32, (G, BN), 0)
    onehot = jnp.where(seg == gid, 1.0, 0.0).astype(jnp.float32)
    contrib = jnp.dot(onehot, h, preferred_element_type=jnp.float32)

    @pl.when(i == 0)
    def _init():
        graph_ref[...] = contrib

    @pl.when(i != 0)
    def _acc():
        graph_ref[...] += contrib

    @pl.when(i == GRID - 1)
    def _head():
        eg = graph_ref[...]
        p = jax.nn.relu(jnp.dot(eg, wp1_ref[...],
                                preferred_element_type=jnp.float32))
        pred_ref[...] = jnp.dot(p, wp2_ref[...],
                                preferred_element_type=jnp.float32) + bp2_ref[...]


def _emb_call(x, W_emb):
    return pl.pallas_call(
        _emb_body,
        grid=(GRID,),
        in_specs=[
            pl.BlockSpec((BN, D_N), lambda i: (i, 0)),
            pl.BlockSpec((D_N, IN_CH), lambda i: (0, 0)),
        ],
        out_specs=pl.BlockSpec((BN, IN_CH), lambda i: (i, 0)),
        out_shape=jax.ShapeDtypeStruct((N, IN_CH), jnp.float32),
    )(x, W_emb)


def _mlp1_call(m1p, W1, b1):
    return pl.pallas_call(
        _mlp1_body,
        grid=(GRID,),
        in_specs=[
            pl.BlockSpec((NC, BN, IN_CH), lambda i: (0, i, 0)),
            pl.BlockSpec((IN_CH, HID), lambda i: (0, 0)),
            pl.BlockSpec((HID,), lambda i: (0,)),
        ],
        out_specs=pl.BlockSpec((NC, BN, EMB), lambda i: (0, i, 0)),
        out_shape=jax.ShapeDtypeStruct((NC, N, EMB), jnp.float32),
    )(m1p, W1, b1)


def _mlp_res_call(m, hprev, W, b):
    return pl.pallas_call(
        _mlp_res_body,
        grid=(GRID,),
        in_specs=[
            pl.BlockSpec((NC, BN, EMB), lambda i: (0, i, 0)),
            pl.BlockSpec((NC, BN, EMB), lambda i: (0, i, 0)),
            pl.BlockSpec((HID, HID), lambda i: (0, 0)),
            pl.BlockSpec((HID,), lambda i: (0,)),
        ],
        out_specs=pl.BlockSpec((NC, BN, EMB), lambda i: (0, i, 0)),
        out_shape=jax.ShapeDtypeStruct((NC, N, EMB), jnp.float32),
    )(m, hprev, W, b)


def _final_call(m3, h2, gid3, W3, b3, Wp1, Wp2, bp2):
    return pl.pallas_call(
        _final_body,
        grid=(GRID,),
        in_specs=[
            pl.BlockSpec((NC, BN, EMB), lambda i: (0, i, 0)),
            pl.BlockSpec((NC, BN, EMB), lambda i: (0, i, 0)),
            pl.BlockSpec((1, 1, BN), lambda i: (i, 0, 0)),
            pl.BlockSpec((HID, HID), lambda i: (0, 0)),
            pl.BlockSpec((HID,), lambda i: (0,)),
            pl.BlockSpec((HID, EMB), lambda i: (0, 0)),
            pl.BlockSpec((EMB, 1), lambda i: (0, 0)),
            pl.BlockSpec((1,), lambda i: (0,)),
        ],
        out_specs=[
            pl.BlockSpec((BN, HID), lambda i: (i, 0)),
            pl.BlockSpec((G, HID), lambda i: (0, 0)),
            pl.BlockSpec((G, 1), lambda i: (0, 0)),
        ],
        out_shape=[
            jax.ShapeDtypeStruct((N, HID), jnp.float32),
            jax.ShapeDtypeStruct((G, HID), jnp.float32),
            jax.ShapeDtypeStruct((G, 1), jnp.float32),
        ],
    )(m3, h2, gid3, W3, b3, Wp1, Wp2, bp2)


@functools.partial(jax.jit)
def kernel(x, edge_index, graph_ids, W_emb, W1, b1, W2, b2, W3, b3, Wp1, Wp2,
           bp2):
    src = edge_index[0]
    dst = edge_index[1]
    pad = E_PAD - E
    srcp = jnp.concatenate([src, jnp.zeros((pad,), jnp.int32)]) \
              .reshape(NSUP, SC_PER_SUP, CH)
    dstp = jnp.concatenate([dst, jnp.full((pad,), N, jnp.int32)]) \
              .reshape(NSUP, SC_PER_SUP, CH)
    zeros8 = jnp.zeros((ZROWS, IN_CH), jnp.float32)
    zeros32 = jnp.zeros((ZROWS, EMB), jnp.float32)
    gid3 = graph_ids.reshape(GRID, 1, BN)

    h0 = _emb_call(x, W_emb)
    m1p = _segsum_d8()(h0, srcp, dstp, zeros8)
    h1 = _mlp1_call(m1p, W1, b1)
    m2 = _segsum_d32()(h1.reshape(NC * N, EMB), srcp, dstp, zeros32)
    h2 = _mlp_res_call(m2, h1, W2, b2)
    m3 = _segsum_d32()(h2.reshape(NC * N, EMB), srcp, dstp, zeros32)
    emb_node, emb_graph, pred = _final_call(m3, h2, gid3, W3, b3, Wp1, Wp2,
                                            bp2)
    return (emb_node, emb_graph, pred)


# speedup vs baseline: 6.2897x; 1.1600x over previous
import functools

import jax
import jax.numpy as jnp
from jax import lax
from jax.experimental import pallas as pl
from jax.experimental.pallas import tpu as pltpu
from jax.experimental.pallas import tpu_sc as plsc

N = 50000
E = 800000
D_N = 35
IN_CH = 8
HID = 64
EMB = 32
G = 64

NC = 2
NS = 16
CH = 128
SC_PER_SUP = 8
NCHUNK = 6400
NSUP = NCHUNK // SC_PER_SUP
E_PAD = NCHUNK * CH
NPAD = 50048
ACC_ROWS = NPAD
ZROWS = ACC_ROWS // NS
OROWS = NPAD // NS
BN = 2000
GRID = N // BN

@functools.cache
def _sc_mesh():
    return plsc.VectorSubcoreMesh(
        core_axis_name="c", subcore_axis_name="s",
        num_cores=NC, num_subcores=NS)


def _seg_body_d8(table, srcp, dstp, zeros, out, sbuf, dbuf, rows, acc,
                 g0, g1, g2, g3, s0, s1, s2, s3):
    gsems = (g0, g1, g2, g3)
    ssems = (s0, s1, s2, s3)
    c = lax.axis_index("c")
    s = lax.axis_index("s")
    pltpu.sync_copy(zeros.at[pl.ds(0, ZROWS)], acc.at[pl.ds(s * ZROWS, ZROWS)])
    plsc.subcore_barrier()

    sups_per_tile = NSUP // NC // NS
    base = c * (NSUP // NC) + s * sups_per_tile

    def step(j, carry):
        q = base + j
        pltpu.sync_copy(srcp.at[q], sbuf)
        pltpu.sync_copy(dstp.at[q], dbuf)
        gps = [pltpu.async_copy(table.at[sbuf.at[r]], rows.at[r % 4],
                                gsems[r % 4])
               for r in range(2)]
        sps = []
        for r in range(SC_PER_SUP):
            gps[r].wait()
            sps.append(pltpu.async_copy(rows.at[r % 4], acc.at[dbuf.at[r]],
                                        ssems[r % 4], add=True))
            if r + 2 < SC_PER_SUP:
                if r - 2 >= 0:
                    sps[r - 2].wait()
                gps.append(pltpu.async_copy(
                    table.at[sbuf.at[r + 2]], rows.at[(r + 2) % 4],
                    gsems[(r + 2) % 4]))
        for r in range(SC_PER_SUP - 4, SC_PER_SUP):
            sps[r].wait()
        return carry

    lax.fori_loop(0, sups_per_tile, step, 0)
    plsc.subcore_barrier()
    pltpu.sync_copy(acc.at[pl.ds(s * OROWS, OROWS)],
                    out.at[c, pl.ds(s * OROWS, OROWS)])


def _seg_body_d32(table, srcp, dstp, zeros, out, sbuf, ibuf, dbuf, rows, acc,
                  g0, g1, g2, g3, s0, s1, s2, s3):
    gsems = (g0, g1, g2, g3)
    ssems = (s0, s1, s2, s3)
    c = lax.axis_index("c")
    s = lax.axis_index("s")
    pltpu.sync_copy(zeros.at[pl.ds(0, ZROWS)], acc.at[pl.ds(s * ZROWS, ZROWS)])
    plsc.subcore_barrier()

    sups_per_tile = NSUP // NS
    base = s * sups_per_tile
    off = c * N

    def step(j, carry):
        q = base + j
        pltpu.sync_copy(srcp.at[q], sbuf)
        pltpu.sync_copy(dstp.at[q], dbuf)
        for r in range(SC_PER_SUP):
            for t in range(CH // 16):
                ibuf[r, pl.ds(t * 16, 16)] = sbuf[r, pl.ds(t * 16, 16)] + off
        gps = [pltpu.async_copy(table.at[ibuf.at[r]], rows.at[r % 4],
                                gsems[r % 4])
               for r in range(2)]
        sps = []
        for r in range(SC_PER_SUP):
            gps[r].wait()
            sps.append(pltpu.async_copy(rows.at[r % 4], acc.at[dbuf.at[r]],
                                        ssems[r % 4], add=True))
            if r + 2 < SC_PER_SUP:
                if r - 2 >= 0:
                    sps[r - 2].wait()
                gps.append(pltpu.async_copy(
                    table.at[ibuf.at[r + 2]], rows.at[(r + 2) % 4],
                    gsems[(r + 2) % 4]))
        for r in range(SC_PER_SUP - 4, SC_PER_SUP):
            sps[r].wait()
        return carry

    lax.fori_loop(0, sups_per_tile, step, 0)
    plsc.subcore_barrier()
    pltpu.sync_copy(acc.at[pl.ds(s * OROWS, OROWS)],
                    out.at[c, pl.ds(s * OROWS, OROWS)])


@functools.cache
def _segsum_d8():
    return pl.kernel(
        _seg_body_d8,
        out_type=jax.ShapeDtypeStruct((NC, NPAD, IN_CH), jnp.float32),
        mesh=_sc_mesh(),
        compiler_params=pltpu.CompilerParams(use_tc_tiling_on_sc=False),
        scratch_types=[
            pltpu.VMEM((SC_PER_SUP, CH), jnp.int32),
            pltpu.VMEM((SC_PER_SUP, CH), jnp.int32),
            pltpu.VMEM((4, CH, IN_CH), jnp.float32),
            pltpu.VMEM_SHARED((ACC_ROWS, IN_CH), jnp.float32),
        ] + [pltpu.SemaphoreType.DMA] * 8 + [
        ],
    )


@functools.cache
def _segsum_d32():
    return pl.kernel(
        _seg_body_d32,
        out_type=jax.ShapeDtypeStruct((NC, NPAD, EMB), jnp.float32),
        mesh=_sc_mesh(),
        compiler_params=pltpu.CompilerParams(use_tc_tiling_on_sc=False),
        scratch_types=[
            pltpu.VMEM((SC_PER_SUP, CH), jnp.int32),
            pltpu.VMEM((SC_PER_SUP, CH), jnp.int32),
            pltpu.VMEM((SC_PER_SUP, CH), jnp.int32),
            pltpu.VMEM((4, CH, EMB), jnp.float32),
            pltpu.VMEM_SHARED((ACC_ROWS, EMB), jnp.float32),
        ] + [pltpu.SemaphoreType.DMA] * 8 + [
        ],
    )


def _emb_body(x_ref, w_ref, o_ref):
    o_ref[...] = jnp.dot(x_ref[...], w_ref[...],
                         preferred_element_type=jnp.float32)


def _mlp1_body(m_ref, w_ref, b_ref, o_ref):
    m = m_ref[0] + m_ref[1]
    h = jax.nn.relu(jnp.dot(m, w_ref[...],
                            preferred_element_type=jnp.float32) + b_ref[...])
    o_ref[0] = h[:, :EMB]
    o_ref[1] = h[:, EMB:]


def _mlp_res_body(m_ref, hprev_ref, w_ref, b_ref, o_ref):
    w = w_ref[...]
    acc = jnp.dot(m_ref[0], w[:EMB], preferred_element_type=jnp.float32)
    acc += jnp.dot(m_ref[1], w[EMB:], preferred_element_type=jnp.float32)
    h = jax.nn.relu(acc + b_ref[...])
    o_ref[0] = h[:, :EMB] + hprev_ref[0]
    o_ref[1] = h[:, EMB:] + hprev_ref[1]


def _final_body(m_ref, hprev_ref, gid_ref, w_ref, b_ref, wp1_ref, wp2_ref,
                bp2_ref, node_ref, graph_ref, pred_ref):
    i = pl.program_id(0)
    w = w_ref[...]
    acc = jnp.dot(m_ref[0], w[:EMB], preferred_element_type=jnp.float32)
    acc += jnp.dot(m_ref[1], w[EMB:], preferred_element_type=jnp.float32)
    h = jax.nn.relu(acc + b_ref[...])
    h = h + jnp.concatenate([hprev_ref[0], hprev_ref[1]], axis=1)
    node_ref[...] = h
    gid = gid_ref[0]
    seg = lax.broadcasted_iota(jnp.int32, (G, BN), 0)
    onehot = jnp.where(seg == gid, 1.0, 0.0).astype(jnp.float32)
    contrib = jnp.dot(onehot, h, preferred_element_type=jnp.float32)

    @pl.when(i == 0)
    def _init():
        graph_ref[...] = contrib

    @pl.when(i != 0)
    def _acc():
        graph_ref[...] += contrib

    @pl.when(i == GRID - 1)
    def _head():
        eg = graph_ref[...]
        p = jax.nn.relu(jnp.dot(eg, wp1_ref[...],
                                preferred_element_type=jnp.float32))
        pred_ref[...] = jnp.dot(p, wp2_ref[...],
                                preferred_element_type=jnp.float32) + bp2_ref[...]


def _emb_call(x, W_emb):
    return pl.pallas_call(
        _emb_body,
        grid=(GRID,),
        in_specs=[
            pl.BlockSpec((BN, D_N), lambda i: (i, 0)),
            pl.BlockSpec((D_N, IN_CH), lambda i: (0, 0)),
        ],
        out_specs=pl.BlockSpec((BN, IN_CH), lambda i: (i, 0)),
        out_shape=jax.ShapeDtypeStruct((N, IN_CH), jnp.float32),
    )(x, W_emb)


def _mlp1_call(m1p, W1, b1):
    return pl.pallas_call(
        _mlp1_body,
        grid=(GRID,),
        in_specs=[
            pl.BlockSpec((NC, BN, IN_CH), lambda i: (0, i, 0)),
            pl.BlockSpec((IN_CH, HID), lambda i: (0, 0)),
            pl.BlockSpec((HID,), lambda i: (0,)),
        ],
        out_specs=pl.BlockSpec((NC, BN, EMB), lambda i: (0, i, 0)),
        out_shape=jax.ShapeDtypeStruct((NC, N, EMB), jnp.float32),
    )(m1p, W1, b1)


def _mlp_res_call(m, hprev, W, b):
    return pl.pallas_call(
        _mlp_res_body,
        grid=(GRID,),
        in_specs=[
            pl.BlockSpec((NC, BN, EMB), lambda i: (0, i, 0)),
            pl.BlockSpec((NC, BN, EMB), lambda i: (0, i, 0)),
            pl.BlockSpec((HID, HID), lambda i: (0, 0)),
            pl.BlockSpec((HID,), lambda i: (0,)),
        ],
        out_specs=pl.BlockSpec((NC, BN, EMB), lambda i: (0, i, 0)),
        out_shape=jax.ShapeDtypeStruct((NC, N, EMB), jnp.float32),
    )(m, hprev, W, b)


def _final_call(m3, h2, gid3, W3, b3, Wp1, Wp2, bp2):
    return pl.pallas_call(
        _final_body,
        grid=(GRID,),
        in_specs=[
            pl.BlockSpec((NC, BN, EMB), lambda i: (0, i, 0)),
            pl.BlockSpec((NC, BN, EMB), lambda i: (0, i, 0)),
            pl.BlockSpec((1, 1, BN), lambda i: (i, 0, 0)),
            pl.BlockSpec((HID, HID), lambda i: (0, 0)),
            pl.BlockSpec((HID,), lambda i: (0,)),
            pl.BlockSpec((HID, EMB), lambda i: (0, 0)),
            pl.BlockSpec((EMB, 1), lambda i: (0, 0)),
            pl.BlockSpec((1,), lambda i: (0,)),
        ],
        out_specs=[
            pl.BlockSpec((BN, HID), lambda i: (i, 0)),
            pl.BlockSpec((G, HID), lambda i: (0, 0)),
            pl.BlockSpec((G, 1), lambda i: (0, 0)),
        ],
        out_shape=[
            jax.ShapeDtypeStruct((N, HID), jnp.float32),
            jax.ShapeDtypeStruct((G, HID), jnp.float32),
            jax.ShapeDtypeStruct((G, 1), jnp.float32),
        ],
    )(m3, h2, gid3, W3, b3, Wp1, Wp2, bp2)


@functools.partial(jax.jit)
def kernel(x, edge_index, graph_ids, W_emb, W1, b1, W2, b2, W3, b3, Wp1, Wp2,
           bp2):
    src = edge_index[0]
    dst = edge_index[1]
    pad = E_PAD - E
    srcp = jnp.concatenate([src, jnp.zeros((pad,), jnp.int32)]) \
              .reshape(NSUP, SC_PER_SUP, CH)
    dstp = jnp.concatenate([dst, jnp.full((pad,), N, jnp.int32)]) \
              .reshape(NSUP, SC_PER_SUP, CH)
    zeros8 = jnp.zeros((ZROWS, IN_CH), jnp.float32)
    zeros32 = jnp.zeros((ZROWS, EMB), jnp.float32)
    gid3 = graph_ids.reshape(GRID, 1, BN)

    h0 = _emb_call(x, W_emb)
    m1p = _segsum_d8()(h0, srcp, dstp, zeros8)
    h1 = _mlp1_call(m1p, W1, b1)
    m2 = _segsum_d32()(h1.reshape(NC * N, EMB), srcp, dstp, zeros32)
    h2 = _mlp_res_call(m2, h1, W2, b2)
    m3 = _segsum_d32()(h2.reshape(NC * N, EMB), srcp, dstp, zeros32)
    emb_node, emb_graph, pred = _final_call(m3, h2, gid3, W3, b3, Wp1, Wp2,
                                            bp2)
    return (emb_node, emb_graph, pred)


# speedup vs baseline: 6.9582x; 1.1063x over previous
import functools

import jax
import jax.numpy as jnp
from jax import lax
from jax.experimental import pallas as pl
from jax.experimental.pallas import tpu as pltpu
from jax.experimental.pallas import tpu_sc as plsc

N = 50000
E = 800000
D_N = 35
IN_CH = 8
HID = 64
EMB = 32
G = 64

NC = 2
NS = 16
CH = 128
SUP = 25
NBUF = 6
DEPTH = 3
NCHUNK = 6400
NSUP = NCHUNK // SUP
E_PAD = NCHUNK * CH
NPAD = 50048
ACC_ROWS = NPAD
ZROWS = ACC_ROWS // NS
OROWS = NPAD // NS
BN = 2000
GRID = N // BN


@functools.cache
def _sc_mesh():
    return plsc.VectorSubcoreMesh(
        core_axis_name="c", subcore_axis_name="s",
        num_cores=NC, num_subcores=NS)


def _edge_pipeline(table, srcp, dstp, acc, sbuf, dbuf, rows, gsems, ssems,
                   first_sup, n_sups):
    def step(j, carry):
        q = first_sup + j
        pltpu.sync_copy(srcp.at[q], sbuf)
        pltpu.sync_copy(dstp.at[q], dbuf)
        gps = [pltpu.async_copy(table.at[sbuf.at[r]], rows.at[r % NBUF],
                                gsems[r % NBUF])
               for r in range(DEPTH)]
        sps = []
        for r in range(SUP):
            gps[r].wait()
            sps.append(pltpu.async_copy(rows.at[r % NBUF], acc.at[dbuf.at[r]],
                                        ssems[r % NBUF], add=True))
            nxt = r + DEPTH
            if nxt < SUP:
                if nxt - NBUF >= 0:
                    sps[nxt - NBUF].wait()
                gps.append(pltpu.async_copy(
                    table.at[sbuf.at[nxt]], rows.at[nxt % NBUF],
                    gsems[nxt % NBUF]))
        for r in range(max(0, SUP - NBUF), SUP):
            sps[r].wait()
        return carry

    lax.fori_loop(0, n_sups, step, 0)


def _seg_body_d8(table, srcp, dstp, zeros, out, sbuf, dbuf, rows, acc, *sems):
    gsems = sems[:NBUF]
    ssems = sems[NBUF:]
    c = lax.axis_index("c")
    s = lax.axis_index("s")
    pltpu.sync_copy(zeros.at[pl.ds(0, ZROWS)], acc.at[pl.ds(s * ZROWS, ZROWS)])
    plsc.subcore_barrier()

    sups_per_tile = NSUP // NC // NS
    first = c * (NSUP // NC) + s * sups_per_tile
    _edge_pipeline(table, srcp, dstp, acc, sbuf, dbuf, rows, gsems, ssems,
                   first, sups_per_tile)
    plsc.subcore_barrier()
    pltpu.sync_copy(acc.at[pl.ds(s * OROWS, OROWS)],
                    out.at[c, pl.ds(s * OROWS, OROWS)])


def _seg_body_d32(table_lo, table_hi, srcp, dstp, zeros, out, sbuf, dbuf,
                  rows, acc, *sems):
    gsems = sems[:NBUF]
    ssems = sems[NBUF:]
    c = lax.axis_index("c")
    s = lax.axis_index("s")
    pltpu.sync_copy(zeros.at[pl.ds(0, ZROWS)], acc.at[pl.ds(s * ZROWS, ZROWS)])
    plsc.subcore_barrier()

    sups_per_tile = NSUP // NS
    first = s * sups_per_tile

    @pl.when(c == 0)
    def _lo():
        _edge_pipeline(table_lo, srcp, dstp, acc, sbuf, dbuf, rows,
                       gsems, ssems, first, sups_per_tile)

    @pl.when(c == 1)
    def _hi():
        _edge_pipeline(table_hi, srcp, dstp, acc, sbuf, dbuf, rows,
                       gsems, ssems, first, sups_per_tile)

    plsc.subcore_barrier()
    pltpu.sync_copy(acc.at[pl.ds(s * OROWS, OROWS)],
                    out.at[c, pl.ds(s * OROWS, OROWS)])


@functools.cache
def _segsum_d8():
    return pl.kernel(
        _seg_body_d8,
        out_type=jax.ShapeDtypeStruct((NC, NPAD, IN_CH), jnp.float32),
        mesh=_sc_mesh(),
        compiler_params=pltpu.CompilerParams(use_tc_tiling_on_sc=False),
        scratch_types=[
            pltpu.VMEM((SUP, CH), jnp.int32),
            pltpu.VMEM((SUP, CH), jnp.int32),
            pltpu.VMEM((NBUF, CH, IN_CH), jnp.float32),
            pltpu.VMEM_SHARED((ACC_ROWS, IN_CH), jnp.float32),
        ] + [pltpu.SemaphoreType.DMA] * (2 * NBUF),
    )


@functools.cache
def _segsum_d32():
    return pl.kernel(
        _seg_body_d32,
        out_type=jax.ShapeDtypeStruct((NC, NPAD, EMB), jnp.float32),
        mesh=_sc_mesh(),
        compiler_params=pltpu.CompilerParams(use_tc_tiling_on_sc=False),
        scratch_types=[
            pltpu.VMEM((SUP, CH), jnp.int32),
            pltpu.VMEM((SUP, CH), jnp.int32),
            pltpu.VMEM((NBUF, CH, EMB), jnp.float32),
            pltpu.VMEM_SHARED((ACC_ROWS, EMB), jnp.float32),
        ] + [pltpu.SemaphoreType.DMA] * (2 * NBUF),
    )


def _emb_body(x_ref, w_ref, o_ref):
    o_ref[...] = jnp.dot(x_ref[...], w_ref[...],
                         preferred_element_type=jnp.float32)


def _mlp1_body(m_ref, w_ref, b_ref, lo_ref, hi_ref):
    m = m_ref[0] + m_ref[1]
    h = jax.nn.relu(jnp.dot(m, w_ref[...],
                            preferred_element_type=jnp.float32) + b_ref[...])
    lo_ref[...] = h[:, :EMB]
    hi_ref[...] = h[:, EMB:]


def _mlp_res_body(m_ref, plo_ref, phi_ref, w_ref, b_ref, lo_ref, hi_ref):
    w = w_ref[...]
    acc = jnp.dot(m_ref[0], w[:EMB], preferred_element_type=jnp.float32)
    acc += jnp.dot(m_ref[1], w[EMB:], preferred_element_type=jnp.float32)
    h = jax.nn.relu(acc + b_ref[...])
    lo_ref[...] = h[:, :EMB] + plo_ref[...]
    hi_ref[...] = h[:, EMB:] + phi_ref[...]


def _final_body(m_ref, plo_ref, phi_ref, gid_ref, w_ref, b_ref, wp1_ref,
                wp2_ref, bp2_ref, node_ref, graph_ref, pred_ref):
    i = pl.program_id(0)
    w = w_ref[...]
    acc = jnp.dot(m_ref[0], w[:EMB], preferred_element_type=jnp.float32)
    acc += jnp.dot(m_ref[1], w[EMB:], preferred_element_type=jnp.float32)
    h = jax.nn.relu(acc + b_ref[...])
    h = h + jnp.concatenate([plo_ref[...], phi_ref[...]], axis=1)
    node_ref[...] = h
    gid = gid_ref[0]
    seg = lax.broadcasted_iota(jnp.int32, (G, BN), 0)
    onehot = jnp.where(seg == gid, 1.0, 0.0).astype(jnp.float32)
    contrib = jnp.dot(onehot, h, preferred_element_type=jnp.float32)

    @pl.when(i == 0)
    def _init():
        graph_ref[...] = contrib

    @pl.when(i != 0)
    def _acc():
        graph_ref[...] += contrib

    @pl.when(i == GRID - 1)
    def _head():
        eg = graph_ref[...]
        p = jax.nn.relu(jnp.dot(eg, wp1_ref[...],
                                preferred_element_type=jnp.float32))
        pred_ref[...] = jnp.dot(p, wp2_ref[...],
                                preferred_element_type=jnp.float32) + bp2_ref[...]


def _emb_call(x, W_emb):
    return pl.pallas_call(
        _emb_body,
        grid=(GRID,),
        in_specs=[
            pl.BlockSpec((BN, D_N), lambda i: (i, 0)),
            pl.BlockSpec((D_N, IN_CH), lambda i: (0, 0)),
        ],
        out_specs=pl.BlockSpec((BN, IN_CH), lambda i: (i, 0)),
        out_shape=jax.ShapeDtypeStruct((N, IN_CH), jnp.float32),
    )(x, W_emb)


def _mlp1_call(m1p, W1, b1):
    return pl.pallas_call(
        _mlp1_body,
        grid=(GRID,),
        in_specs=[
            pl.BlockSpec((NC, BN, IN_CH), lambda i: (0, i, 0)),
            pl.BlockSpec((IN_CH, HID), lambda i: (0, 0)),
            pl.BlockSpec((HID,), lambda i: (0,)),
        ],
        out_specs=[
            pl.BlockSpec((BN, EMB), lambda i: (i, 0)),
            pl.BlockSpec((BN, EMB), lambda i: (i, 0)),
        ],
        out_shape=[
            jax.ShapeDtypeStruct((N, EMB), jnp.float32),
            jax.ShapeDtypeStruct((N, EMB), jnp.float32),
        ],
    )(m1p, W1, b1)


def _mlp_res_call(m, plo, phi, W, b):
    return pl.pallas_call(
        _mlp_res_body,
        grid=(GRID,),
        in_specs=[
            pl.BlockSpec((NC, BN, EMB), lambda i: (0, i, 0)),
            pl.BlockSpec((BN, EMB), lambda i: (i, 0)),
            pl.BlockSpec((BN, EMB), lambda i: (i, 0)),
            pl.BlockSpec((HID, HID), lambda i: (0, 0)),
            pl.BlockSpec((HID,), lambda i: (0,)),
        ],
        out_specs=[
            pl.BlockSpec((BN, EMB), lambda i: (i, 0)),
            pl.BlockSpec((BN, EMB), lambda i: (i, 0)),
        ],
        out_shape=[
            jax.ShapeDtypeStruct((N, EMB), jnp.float32),
            jax.ShapeDtypeStruct((N, EMB), jnp.float32),
        ],
    )(m, plo, phi, W, b)


def _final_call(m3, plo, phi, gid3, W3, b3, Wp1, Wp2, bp2):
    return pl.pallas_call(
        _final_body,
        grid=(GRID,),
        in_specs=[
            pl.BlockSpec((NC, BN, EMB), lambda i: (0, i, 0)),
            pl.BlockSpec((BN, EMB), lambda i: (i, 0)),
            pl.BlockSpec((BN, EMB), lambda i: (i, 0)),
            pl.BlockSpec((1, 1, BN), lambda i: (i, 0, 0)),
            pl.BlockSpec((HID, HID), lambda i: (0, 0)),
            pl.BlockSpec((HID,), lambda i: (0,)),
            pl.BlockSpec((HID, EMB), lambda i: (0, 0)),
            pl.BlockSpec((EMB, 1), lambda i: (0, 0)),
            pl.BlockSpec((1,), lambda i: (0,)),
        ],
        out_specs=[
            pl.BlockSpec((BN, HID), lambda i: (i, 0)),
            pl.BlockSpec((G, HID), lambda i: (0, 0)),
            pl.BlockSpec((G, 1), lambda i: (0, 0)),
        ],
        out_shape=[
            jax.ShapeDtypeStruct((N, HID), jnp.float32),
            jax.ShapeDtypeStruct((G, HID), jnp.float32),
            jax.ShapeDtypeStruct((G, 1), jnp.float32),
        ],
    )(m3, plo, phi, gid3, W3, b3, Wp1, Wp2, bp2)


@functools.partial(jax.jit)
def kernel(x, edge_index, graph_ids, W_emb, W1, b1, W2, b2, W3, b3, Wp1, Wp2,
           bp2):
    src = edge_index[0]
    dst = edge_index[1]
    pad = E_PAD - E
    srcp = jnp.concatenate([src, jnp.zeros((pad,), jnp.int32)]) \
              .reshape(NSUP, SUP, CH)
    dstp = jnp.concatenate([dst, jnp.full((pad,), N, jnp.int32)]) \
              .reshape(NSUP, SUP, CH)
    zeros8 = jnp.zeros((ZROWS, IN_CH), jnp.float32)
    zeros32 = jnp.zeros((ZROWS, EMB), jnp.float32)
    gid3 = graph_ids.reshape(GRID, 1, BN)

    h0 = _emb_call(x, W_emb)
    m1p = _segsum_d8()(h0, srcp, dstp, zeros8)
    h1lo, h1hi = _mlp1_call(m1p, W1, b1)
    m2 = _segsum_d32()(h1lo, h1hi, srcp, dstp, zeros32)
    h2lo, h2hi = _mlp_res_call(m2, h1lo, h1hi, W2, b2)
    m3 = _segsum_d32()(h2lo, h2hi, srcp, dstp, zeros32)
    emb_node, emb_graph, pred = _final_call(m3, h2lo, h2hi, gid3, W3, b3,
                                            Wp1, Wp2, bp2)
    return (emb_node, emb_graph, pred)


# speedup vs baseline: 7.1298x; 1.0247x over previous
import functools

import jax
import jax.numpy as jnp
from jax import lax
from jax.experimental import pallas as pl
from jax.experimental.pallas import tpu as pltpu
from jax.experimental.pallas import tpu_sc as plsc

N = 50000
E = 800000
D_N = 35
IN_CH = 8
HID = 64
EMB = 32
G = 64

NC = 2
NS = 16
CH = 128
SUP = 25
NBUF = 4
DEPTH = 2
NF = 3
NCHUNK = 6400
NSUP = NCHUNK // SUP
E_PAD = NCHUNK * CH
NPAD = 50048
ACC_ROWS = NPAD
ZROWS = ACC_ROWS // NS
OROWS = NPAD // NS
BN = 2000
GRID = N // BN


@functools.cache
def _sc_mesh():
    return plsc.VectorSubcoreMesh(
        core_axis_name="c", subcore_axis_name="s",
        num_cores=NC, num_subcores=NS)


def _edge_pipeline(table, srcp, dstp, acc, sbuf, dbuf, rows, gsems, ssems,
                   first_sup, n_sups, frows=None, nf=NF):
    packed = frows is not None
    UNROLL = 8

    def expand(gslot, fslot):
        def conv(i, carry):
            for u in range(UNROLL):
                v = rows[gslot, i * UNROLL + u, :]
                lo = plsc.bitcast(v << 16, jnp.float32)
                hi = plsc.bitcast(v & jnp.int32(-65536), jnp.float32)
                frows[fslot, i * UNROLL + u, pl.ds(0, 16)] = lo
                frows[fslot, i * UNROLL + u, pl.ds(16, 16)] = hi
            return carry
        lax.fori_loop(0, CH // UNROLL, conv, 0)

    def step(j, carry):
        q = first_sup + j
        pltpu.sync_copy(srcp.at[q], sbuf)
        pltpu.sync_copy(dstp.at[q], dbuf)
        gps = [pltpu.async_copy(table.at[sbuf.at[r]], rows.at[r % NBUF],
                                gsems[r % NBUF])
               for r in range(DEPTH)]
        sps = []
        for r in range(SUP):
            gps[r].wait()
            if r - nf >= 0:
                sps[r - nf].wait()
            if packed:
                expand(r % NBUF, r % nf)
                src_buf = frows.at[r % nf]
            else:
                src_buf = rows.at[r % NBUF]
            sps.append(pltpu.async_copy(src_buf, acc.at[dbuf.at[r]],
                                        ssems[r % nf], add=True))
            nxt = r + DEPTH
            if nxt < SUP:
                gps.append(pltpu.async_copy(
                    table.at[sbuf.at[nxt]], rows.at[nxt % NBUF],
                    gsems[nxt % NBUF]))
        for r in range(max(0, SUP - nf), SUP):
            sps[r].wait()
        return carry

    lax.fori_loop(0, n_sups, step, 0)


def _seg_body_d8(table, srcp, dstp, zeros, out, sbuf, dbuf, rows, acc, *sems):
    gsems = sems[:NBUF]
    ssems = sems[NBUF:]
    c = lax.axis_index("c")
    s = lax.axis_index("s")
    pltpu.sync_copy(zeros.at[pl.ds(0, ZROWS)], acc.at[pl.ds(s * ZROWS, ZROWS)])
    plsc.subcore_barrier()

    sups_per_tile = NSUP // NC // NS
    first = c * (NSUP // NC) + s * sups_per_tile
    _edge_pipeline(table, srcp, dstp, acc, sbuf, dbuf, rows, gsems, ssems,
                   first, sups_per_tile, nf=2)
    plsc.subcore_barrier()
    pltpu.sync_copy(acc.at[pl.ds(s * OROWS, OROWS)],
                    out.at[c, pl.ds(s * OROWS, OROWS)])


def _seg_body_d32(table_lo, table_hi, srcp, dstp, zeros, out, sbuf, dbuf,
                  rows, frows, acc, *sems):
    gsems = sems[:NBUF]
    ssems = sems[NBUF:]
    c = lax.axis_index("c")
    s = lax.axis_index("s")
    pltpu.sync_copy(zeros.at[pl.ds(0, ZROWS)], acc.at[pl.ds(s * ZROWS, ZROWS)])
    plsc.subcore_barrier()

    sups_per_tile = NSUP // NS
    first = s * sups_per_tile

    @pl.when(c == 0)
    def _lo():
        _edge_pipeline(table_lo, srcp, dstp, acc, sbuf, dbuf, rows,
                       gsems, ssems, first, sups_per_tile, frows)

    @pl.when(c == 1)
    def _hi():
        _edge_pipeline(table_hi, srcp, dstp, acc, sbuf, dbuf, rows,
                       gsems, ssems, first, sups_per_tile, frows)

    plsc.subcore_barrier()
    pltpu.sync_copy(acc.at[pl.ds(s * OROWS, OROWS)],
                    out.at[c, pl.ds(s * OROWS, OROWS)])


@functools.cache
def _segsum_d8():
    return pl.kernel(
        _seg_body_d8,
        out_type=jax.ShapeDtypeStruct((NC, NPAD, IN_CH), jnp.float32),
        mesh=_sc_mesh(),
        compiler_params=pltpu.CompilerParams(use_tc_tiling_on_sc=False),
        scratch_types=[
            pltpu.VMEM((SUP, CH), jnp.int32),
            pltpu.VMEM((SUP, CH), jnp.int32),
            pltpu.VMEM((NBUF, CH, IN_CH), jnp.float32),
            pltpu.VMEM_SHARED((ACC_ROWS, IN_CH), jnp.float32),
        ] + [pltpu.SemaphoreType.DMA] * (NBUF + 2),
    )


@functools.cache
def _segsum_d32():
    return pl.kernel(
        _seg_body_d32,
        out_type=jax.ShapeDtypeStruct((NC, NPAD, EMB), jnp.float32),
        mesh=_sc_mesh(),
        compiler_params=pltpu.CompilerParams(use_tc_tiling_on_sc=False,
                                             needs_layout_passes=False),
        scratch_types=[
            pltpu.VMEM((SUP, CH), jnp.int32),
            pltpu.VMEM((SUP, CH), jnp.int32),
            pltpu.VMEM((NBUF, CH, EMB // 2), jnp.int32),
            pltpu.VMEM((NF, CH, EMB), jnp.float32),
            pltpu.VMEM_SHARED((ACC_ROWS, EMB), jnp.float32),
        ] + [pltpu.SemaphoreType.DMA] * (NBUF + NF),
    )


def _emb_body(x_ref, w_ref, o_ref):
    o_ref[...] = jnp.dot(x_ref[...], w_ref[...],
                         preferred_element_type=jnp.float32)


def _pack_half(h):
    a = lax.bitcast_convert_type(h[:, :16].astype(jnp.bfloat16),
                                 jnp.uint16).astype(jnp.uint32)
    b = lax.bitcast_convert_type(h[:, 16:].astype(jnp.bfloat16),
                                 jnp.uint16).astype(jnp.uint32)
    return lax.bitcast_convert_type((b << 16) | a, jnp.int32)


def _mlp1_body(m_ref, w_ref, b_ref, lo_ref, hi_ref, blo_ref, bhi_ref):
    m = m_ref[0] + m_ref[1]
    h = jax.nn.relu(jnp.dot(m, w_ref[...],
                            preferred_element_type=jnp.float32) + b_ref[...])
    lo_ref[...] = h[:, :EMB]
    hi_ref[...] = h[:, EMB:]
    blo_ref[...] = _pack_half(h[:, :EMB])
    bhi_ref[...] = _pack_half(h[:, EMB:])


def _mlp_res_body(m_ref, plo_ref, phi_ref, w_ref, b_ref, lo_ref, hi_ref,
                  blo_ref, bhi_ref):
    w = w_ref[...]
    acc = jnp.dot(m_ref[0], w[:EMB], preferred_element_type=jnp.float32)
    acc += jnp.dot(m_ref[1], w[EMB:], preferred_element_type=jnp.float32)
    h = jax.nn.relu(acc + b_ref[...])
    lo = h[:, :EMB] + plo_ref[...]
    hi = h[:, EMB:] + phi_ref[...]
    lo_ref[...] = lo
    hi_ref[...] = hi
    blo_ref[...] = _pack_half(lo)
    bhi_ref[...] = _pack_half(hi)


def _final_body(m_ref, plo_ref, phi_ref, gid_ref, w_ref, b_ref, wp1_ref,
                wp2_ref, bp2_ref, node_ref, graph_ref, pred_ref):
    i = pl.program_id(0)
    w = w_ref[...]
    acc = jnp.dot(m_ref[0], w[:EMB], preferred_element_type=jnp.float32)
    acc += jnp.dot(m_ref[1], w[EMB:], preferred_element_type=jnp.float32)
    h = jax.nn.relu(acc + b_ref[...])
    h = h + jnp.concatenate([plo_ref[...], phi_ref[...]], axis=1)
    node_ref[...] = h
    gid = gid_ref[0]
    seg = lax.broadcasted_iota(jnp.int32, (G, BN), 0)
    onehot = jnp.where(seg == gid, 1.0, 0.0).astype(jnp.float32)
    contrib = jnp.dot(onehot, h, preferred_element_type=jnp.float32)

    @pl.when(i == 0)
    def _init():
        graph_ref[...] = contrib

    @pl.when(i != 0)
    def _acc():
        graph_ref[...] += contrib

    @pl.when(i == GRID - 1)
    def _head():
        eg = graph_ref[...]
        p = jax.nn.relu(jnp.dot(eg, wp1_ref[...],
                                preferred_element_type=jnp.float32))
        pred_ref[...] = jnp.dot(p, wp2_ref[...],
                                preferred_element_type=jnp.float32) + bp2_ref[...]


def _emb_call(x, W_emb):
    return pl.pallas_call(
        _emb_body,
        grid=(GRID,),
        in_specs=[
            pl.BlockSpec((BN, D_N), lambda i: (i, 0)),
            pl.BlockSpec((D_N, IN_CH), lambda i: (0, 0)),
        ],
        out_specs=pl.BlockSpec((BN, IN_CH), lambda i: (i, 0)),
        out_shape=jax.ShapeDtypeStruct((N, IN_CH), jnp.float32),
    )(x, W_emb)


def _mlp1_call(m1p, W1, b1):
    return pl.pallas_call(
        _mlp1_body,
        grid=(GRID,),
        in_specs=[
            pl.BlockSpec((NC, BN, IN_CH), lambda i: (0, i, 0)),
            pl.BlockSpec((IN_CH, HID), lambda i: (0, 0)),
            pl.BlockSpec((HID,), lambda i: (0,)),
        ],
        out_specs=[
            pl.BlockSpec((BN, EMB), lambda i: (i, 0)),
            pl.BlockSpec((BN, EMB), lambda i: (i, 0)),
            pl.BlockSpec((BN, EMB // 2), lambda i: (i, 0)),
            pl.BlockSpec((BN, EMB // 2), lambda i: (i, 0)),
        ],
        out_shape=[
            jax.ShapeDtypeStruct((N, EMB), jnp.float32),
            jax.ShapeDtypeStruct((N, EMB), jnp.float32),
            jax.ShapeDtypeStruct((N, EMB // 2), jnp.int32),
            jax.ShapeDtypeStruct((N, EMB // 2), jnp.int32),
        ],
    )(m1p, W1, b1)


def _mlp_res_call(m, plo, phi, W, b):
    return pl.pallas_call(
        _mlp_res_body,
        grid=(GRID,),
        in_specs=[
            pl.BlockSpec((NC, BN, EMB), lambda i: (0, i, 0)),
            pl.BlockSpec((BN, EMB), lambda i: (i, 0)),
            pl.BlockSpec((BN, EMB), lambda i: (i, 0)),
            pl.BlockSpec((HID, HID), lambda i: (0, 0)),
            pl.BlockSpec((HID,), lambda i: (0,)),
        ],
        out_specs=[
            pl.BlockSpec((BN, EMB), lambda i: (i, 0)),
            pl.BlockSpec((BN, EMB), lambda i: (i, 0)),
            pl.BlockSpec((BN, EMB // 2), lambda i: (i, 0)),
            pl.BlockSpec((BN, EMB // 2), lambda i: (i, 0)),
        ],
        out_shape=[
            jax.ShapeDtypeStruct((N, EMB), jnp.float32),
            jax.ShapeDtypeStruct((N, EMB), jnp.float32),
            jax.ShapeDtypeStruct((N, EMB // 2), jnp.int32),
            jax.ShapeDtypeStruct((N, EMB // 2), jnp.int32),
        ],
    )(m, plo, phi, W, b)


def _final_call(m3, plo, phi, gid3, W3, b3, Wp1, Wp2, bp2):
    return pl.pallas_call(
        _final_body,
        grid=(GRID,),
        in_specs=[
            pl.BlockSpec((NC, BN, EMB), lambda i: (0, i, 0)),
            pl.BlockSpec((BN, EMB), lambda i: (i, 0)),
            pl.BlockSpec((BN, EMB), lambda i: (i, 0)),
            pl.BlockSpec((1, 1, BN), lambda i: (i, 0, 0)),
            pl.BlockSpec((HID, HID), lambda i: (0, 0)),
            pl.BlockSpec((HID,), lambda i: (0,)),
            pl.BlockSpec((HID, EMB), lambda i: (0, 0)),
            pl.BlockSpec((EMB, 1), lambda i: (0, 0)),
            pl.BlockSpec((1,), lambda i: (0,)),
        ],
        out_specs=[
            pl.BlockSpec((BN, HID), lambda i: (i, 0)),
            pl.BlockSpec((G, HID), lambda i: (0, 0)),
            pl.BlockSpec((G, 1), lambda i: (0, 0)),
        ],
        out_shape=[
            jax.ShapeDtypeStruct((N, HID), jnp.float32),
            jax.ShapeDtypeStruct((G, HID), jnp.float32),
            jax.ShapeDtypeStruct((G, 1), jnp.float32),
        ],
    )(m3, plo, phi, gid3, W3, b3, Wp1, Wp2, bp2)


@functools.partial(jax.jit)
def kernel(x, edge_index, graph_ids, W_emb, W1, b1, W2, b2, W3, b3, Wp1, Wp2,
           bp2):
    src = edge_index[0]
    dst = edge_index[1]
    pad = E_PAD - E
    srcp = jnp.concatenate([src, jnp.zeros((pad,), jnp.int32)]) \
              .reshape(NSUP, SUP, CH)
    dstp = jnp.concatenate([dst, jnp.full((pad,), N, jnp.int32)]) \
              .reshape(NSUP, SUP, CH)
    zeros8 = jnp.zeros((ZROWS, IN_CH), jnp.float32)
    zeros32 = jnp.zeros((ZROWS, EMB), jnp.float32)
    gid3 = graph_ids.reshape(GRID, 1, BN)

    h0 = _emb_call(x, W_emb)
    m1p = _segsum_d8()(h0, srcp, dstp, zeros8)
    h1lo, h1hi, b1lo, b1hi = _mlp1_call(m1p, W1, b1)
    m2 = _segsum_d32()(b1lo, b1hi, srcp, dstp, zeros32)
    h2lo, h2hi, b2lo, b2hi = _mlp_res_call(m2, h1lo, h1hi, W2, b2)
    m3 = _segsum_d32()(b2lo, b2hi, srcp, dstp, zeros32)
    emb_node, emb_graph, pred = _final_call(m3, h2lo, h2hi, gid3, W3, b3,
                                            Wp1, Wp2, bp2)
    return (emb_node, emb_graph, pred)


# speedup vs baseline: 8.8406x; 1.2399x over previous
import functools

import jax
import jax.numpy as jnp
from jax import lax
from jax.experimental import pallas as pl
from jax.experimental.pallas import tpu as pltpu
from jax.experimental.pallas import tpu_sc as plsc

N = 50000
E = 800000
D_N = 35
IN_CH = 8
HID = 64
EMB = 32
G = 64

NC = 2
NS = 16
CH = 128
SUP = 25
NBUF = 6
DEPTH = 4
NF = 3
NSUP = E // (SUP * CH)
NPAD = 50048
ACC_ROWS = NPAD
ZROWS = ACC_ROWS // NS
OROWS = NPAD // NS
BN = 2000
GRID = N // BN


@functools.cache
def _sc_mesh():
    return plsc.VectorSubcoreMesh(
        core_axis_name="c", subcore_axis_name="s",
        num_cores=NC, num_subcores=NS)


def _edge_pipeline(table, ei, acc, sbuf, dbuf, rows, gsems, ssems,
                   first_sup, n_sups, frows=None, nf=NF):
    packed = frows is not None
    UNROLL = 8

    def expand(gslot, fslot):
        def conv(i, carry):
            for u in range(UNROLL):
                v = rows[gslot, i * UNROLL + u, :]
                lo = plsc.bitcast(v << 16, jnp.float32)
                hi = plsc.bitcast(v & jnp.int32(-65536), jnp.float32)
                frows[fslot, i * UNROLL + u, pl.ds(0, 16)] = lo
                frows[fslot, i * UNROLL + u, pl.ds(16, 16)] = hi
            return carry
        lax.fori_loop(0, CH // UNROLL, conv, 0)

    def step(j, carry):
        q = first_sup + j
        pltpu.sync_copy(ei.at[0, q], sbuf)
        pltpu.sync_copy(ei.at[1, q], dbuf)
        gps = [pltpu.async_copy(table.at[sbuf.at[r]], rows.at[r % NBUF],
                                gsems[r % NBUF])
               for r in range(DEPTH)]
        sps = []
        for r in range(SUP):
            gps[r].wait()
            if r - nf >= 0:
                sps[r - nf].wait()
            if packed:
                expand(r % NBUF, r % nf)
                src_buf = frows.at[r % nf]
            else:
                src_buf = rows.at[r % NBUF]
            sps.append(pltpu.async_copy(src_buf, acc.at[dbuf.at[r]],
                                        ssems[r % nf], add=True))
            nxt = r + DEPTH
            if nxt < SUP:
                gps.append(pltpu.async_copy(
                    table.at[sbuf.at[nxt]], rows.at[nxt % NBUF],
                    gsems[nxt % NBUF]))
        for r in range(max(0, SUP - nf), SUP):
            sps[r].wait()
        return carry

    lax.fori_loop(0, n_sups, step, 0)


def _seg_body_d8(table, ei, zeros, out, sbuf, dbuf, rows, acc, *sems):
    gsems = sems[:NBUF]
    ssems = sems[NBUF:]
    c = lax.axis_index("c")
    s = lax.axis_index("s")
    pltpu.sync_copy(zeros.at[pl.ds(0, ZROWS)], acc.at[pl.ds(s * ZROWS, ZROWS)])
    plsc.subcore_barrier()

    half = NSUP // NC
    n_sups = jnp.where(s < 13, 8, 7)
    first = c * half + 7 * s + jnp.minimum(s, 13)
    _edge_pipeline(table, ei, acc, sbuf, dbuf, rows, gsems, ssems,
                   first, n_sups, nf=2)
    plsc.subcore_barrier()
    pltpu.sync_copy(acc.at[pl.ds(s * OROWS, OROWS)],
                    out.at[c, pl.ds(s * OROWS, OROWS)])


def _seg_body_d32(table_lo, table_hi, ei, zeros, out, sbuf, dbuf,
                  rows, frows, acc, *sems):
    gsems = sems[:NBUF]
    ssems = sems[NBUF:]
    c = lax.axis_index("c")
    s = lax.axis_index("s")
    pltpu.sync_copy(zeros.at[pl.ds(0, ZROWS)], acc.at[pl.ds(s * ZROWS, ZROWS)])
    plsc.subcore_barrier()

    n_sups = jnp.where(s < 10, 16, 15)
    first = 15 * s + jnp.minimum(s, 10)

    @pl.when(c == 0)
    def _lo():
        _edge_pipeline(table_lo, ei, acc, sbuf, dbuf, rows,
                       gsems, ssems, first, n_sups, frows)

    @pl.when(c == 1)
    def _hi():
        _edge_pipeline(table_hi, ei, acc, sbuf, dbuf, rows,
                       gsems, ssems, first, n_sups, frows)

    plsc.subcore_barrier()
    pltpu.sync_copy(acc.at[pl.ds(s * OROWS, OROWS)],
                    out.at[c, pl.ds(s * OROWS, OROWS)])


@functools.cache
def _segsum_d8():
    return pl.kernel(
        _seg_body_d8,
        out_type=jax.ShapeDtypeStruct((NC, NPAD, IN_CH), jnp.float32),
        mesh=_sc_mesh(),
        compiler_params=pltpu.CompilerParams(use_tc_tiling_on_sc=False),
        scratch_types=[
            pltpu.VMEM((SUP, CH), jnp.int32),
            pltpu.VMEM((SUP, CH), jnp.int32),
            pltpu.VMEM((NBUF, CH, IN_CH), jnp.float32),
            pltpu.VMEM_SHARED((ACC_ROWS, IN_CH), jnp.float32),
        ] + [pltpu.SemaphoreType.DMA] * (NBUF + 2),
    )


@functools.cache
def _segsum_d32():
    return pl.kernel(
        _seg_body_d32,
        out_type=jax.ShapeDtypeStruct((NC, NPAD, EMB), jnp.float32),
        mesh=_sc_mesh(),
        compiler_params=pltpu.CompilerParams(use_tc_tiling_on_sc=False,
                                             needs_layout_passes=False),
        scratch_types=[
            pltpu.VMEM((SUP, CH), jnp.int32),
            pltpu.VMEM((SUP, CH), jnp.int32),
            pltpu.VMEM((NBUF, CH, EMB // 2), jnp.int32),
            pltpu.VMEM((NF, CH, EMB), jnp.float32),
            pltpu.VMEM_SHARED((ACC_ROWS, EMB), jnp.float32),
        ] + [pltpu.SemaphoreType.DMA] * (NBUF + NF),
    )


def _emb_body(x_ref, w_ref, o_ref):
    o_ref[...] = jnp.dot(x_ref[...], w_ref[...],
                         preferred_element_type=jnp.float32)


def _pack_half(h):
    a = lax.bitcast_convert_type(h[:, :16].astype(jnp.bfloat16),
                                 jnp.uint16).astype(jnp.uint32)
    b = lax.bitcast_convert_type(h[:, 16:].astype(jnp.bfloat16),
                                 jnp.uint16).astype(jnp.uint32)
    return lax.bitcast_convert_type((b << 16) | a, jnp.int32)


def _mlp1_body(m_ref, w_ref, b_ref, lo_ref, hi_ref, blo_ref, bhi_ref):
    m = m_ref[0] + m_ref[1]
    h = jax.nn.relu(jnp.dot(m, w_ref[...],
                            preferred_element_type=jnp.float32) + b_ref[...])
    lo_ref[...] = h[:, :EMB]
    hi_ref[...] = h[:, EMB:]
    blo_ref[...] = _pack_half(h[:, :EMB])
    bhi_ref[...] = _pack_half(h[:, EMB:])


def _mlp_res_body(m_ref, plo_ref, phi_ref, w_ref, b_ref, lo_ref, hi_ref,
                  blo_ref, bhi_ref):
    w = w_ref[...]
    acc = jnp.dot(m_ref[0], w[:EMB], preferred_element_type=jnp.float32)
    acc += jnp.dot(m_ref[1], w[EMB:], preferred_element_type=jnp.float32)
    h = jax.nn.relu(acc + b_ref[...])
    lo = h[:, :EMB] + plo_ref[...]
    hi = h[:, EMB:] + phi_ref[...]
    lo_ref[...] = lo
    hi_ref[...] = hi
    blo_ref[...] = _pack_half(lo)
    bhi_ref[...] = _pack_half(hi)


def _final_body(m_ref, plo_ref, phi_ref, gid_ref, w_ref, b_ref, wp1_ref,
                wp2_ref, bp2_ref, node_ref, graph_ref, pred_ref):
    i = pl.program_id(0)
    w = w_ref[...]
    acc = jnp.dot(m_ref[0], w[:EMB], preferred_element_type=jnp.float32)
    acc += jnp.dot(m_ref[1], w[EMB:], preferred_element_type=jnp.float32)
    h = jax.nn.relu(acc + b_ref[...])
    h = h + jnp.concatenate([plo_ref[...], phi_ref[...]], axis=1)
    node_ref[...] = h
    gid = gid_ref[0]
    seg = lax.broadcasted_iota(jnp.int32, (G, BN), 0)
    onehot = jnp.where(seg == gid, 1.0, 0.0).astype(jnp.float32)
    contrib = jnp.dot(onehot, h, preferred_element_type=jnp.float32)

    @pl.when(i == 0)
    def _init():
        graph_ref[...] = contrib

    @pl.when(i != 0)
    def _acc():
        graph_ref[...] += contrib

    @pl.when(i == GRID - 1)
    def _head():
        eg = graph_ref[...]
        p = jax.nn.relu(jnp.dot(eg, wp1_ref[...],
                                preferred_element_type=jnp.float32))
        pred_ref[...] = jnp.dot(p, wp2_ref[...],
                                preferred_element_type=jnp.float32) + bp2_ref[...]


def _emb_call(x, W_emb):
    return pl.pallas_call(
        _emb_body,
        grid=(GRID,),
        in_specs=[
            pl.BlockSpec((BN, D_N), lambda i: (i, 0)),
            pl.BlockSpec((D_N, IN_CH), lambda i: (0, 0)),
        ],
        out_specs=pl.BlockSpec((BN, IN_CH), lambda i: (i, 0)),
        out_shape=jax.ShapeDtypeStruct((N, IN_CH), jnp.float32),
    )(x, W_emb)


def _mlp1_call(m1p, W1, b1):
    return pl.pallas_call(
        _mlp1_body,
        grid=(GRID,),
        in_specs=[
            pl.BlockSpec((NC, BN, IN_CH), lambda i: (0, i, 0)),
            pl.BlockSpec((IN_CH, HID), lambda i: (0, 0)),
            pl.BlockSpec((HID,), lambda i: (0,)),
        ],
        out_specs=[
            pl.BlockSpec((BN, EMB), lambda i: (i, 0)),
            pl.BlockSpec((BN, EMB), lambda i: (i, 0)),
            pl.BlockSpec((BN, EMB // 2), lambda i: (i, 0)),
            pl.BlockSpec((BN, EMB // 2), lambda i: (i, 0)),
        ],
        out_shape=[
            jax.ShapeDtypeStruct((N, EMB), jnp.float32),
            jax.ShapeDtypeStruct((N, EMB), jnp.float32),
            jax.ShapeDtypeStruct((N, EMB // 2), jnp.int32),
            jax.ShapeDtypeStruct((N, EMB // 2), jnp.int32),
        ],
    )(m1p, W1, b1)


def _mlp_res_call(m, plo, phi, W, b):
    return pl.pallas_call(
        _mlp_res_body,
        grid=(GRID,),
        in_specs=[
            pl.BlockSpec((NC, BN, EMB), lambda i: (0, i, 0)),
            pl.BlockSpec((BN, EMB), lambda i: (i, 0)),
            pl.BlockSpec((BN, EMB), lambda i: (i, 0)),
            pl.BlockSpec((HID, HID), lambda i: (0, 0)),
            pl.BlockSpec((HID,), lambda i: (0,)),
        ],
        out_specs=[
            pl.BlockSpec((BN, EMB), lambda i: (i, 0)),
            pl.BlockSpec((BN, EMB), lambda i: (i, 0)),
            pl.BlockSpec((BN, EMB // 2), lambda i: (i, 0)),
            pl.BlockSpec((BN, EMB // 2), lambda i: (i, 0)),
        ],
        out_shape=[
            jax.ShapeDtypeStruct((N, EMB), jnp.float32),
            jax.ShapeDtypeStruct((N, EMB), jnp.float32),
            jax.ShapeDtypeStruct((N, EMB // 2), jnp.int32),
            jax.ShapeDtypeStruct((N, EMB // 2), jnp.int32),
        ],
    )(m, plo, phi, W, b)


def _final_call(m3, plo, phi, gid3, W3, b3, Wp1, Wp2, bp2):
    return pl.pallas_call(
        _final_body,
        grid=(GRID,),
        in_specs=[
            pl.BlockSpec((NC, BN, EMB), lambda i: (0, i, 0)),
            pl.BlockSpec((BN, EMB), lambda i: (i, 0)),
            pl.BlockSpec((BN, EMB), lambda i: (i, 0)),
            pl.BlockSpec((1, 1, BN), lambda i: (i, 0, 0)),
            pl.BlockSpec((HID, HID), lambda i: (0, 0)),
            pl.BlockSpec((HID,), lambda i: (0,)),
            pl.BlockSpec((HID, EMB), lambda i: (0, 0)),
            pl.BlockSpec((EMB, 1), lambda i: (0, 0)),
            pl.BlockSpec((1,), lambda i: (0,)),
        ],
        out_specs=[
            pl.BlockSpec((BN, HID), lambda i: (i, 0)),
            pl.BlockSpec((G, HID), lambda i: (0, 0)),
            pl.BlockSpec((G, 1), lambda i: (0, 0)),
        ],
        out_shape=[
            jax.ShapeDtypeStruct((N, HID), jnp.float32),
            jax.ShapeDtypeStruct((G, HID), jnp.float32),
            jax.ShapeDtypeStruct((G, 1), jnp.float32),
        ],
    )(m3, plo, phi, gid3, W3, b3, Wp1, Wp2, bp2)


@functools.partial(jax.jit)
def kernel(x, edge_index, graph_ids, W_emb, W1, b1, W2, b2, W3, b3, Wp1, Wp2,
           bp2):
    ei = edge_index.reshape(NC, NSUP, SUP, CH)
    zeros8 = jnp.zeros((ZROWS, IN_CH), jnp.float32)
    zeros32 = jnp.zeros((ZROWS, EMB), jnp.float32)
    gid3 = graph_ids.reshape(GRID, 1, BN)

    h0 = _emb_call(x, W_emb)
    m1p = _segsum_d8()(h0, ei, zeros8)
    h1lo, h1hi, b1lo, b1hi = _mlp1_call(m1p, W1, b1)
    m2 = _segsum_d32()(b1lo, b1hi, ei, zeros32)
    h2lo, h2hi, b2lo, b2hi = _mlp_res_call(m2, h1lo, h1hi, W2, b2)
    m3 = _segsum_d32()(b2lo, b2hi, ei, zeros32)
    emb_node, emb_graph, pred = _final_call(m3, h2lo, h2hi, gid3, W3, b3,
                                            Wp1, Wp2, bp2)
    return (emb_node, emb_graph, pred)
